# baseline (speedup 1.0000x reference)
import jax
import jax.numpy as jnp
from jax import lax
from jax.experimental import pallas as pl
from jax.experimental.pallas import tpu as pltpu
from jax.experimental.pallas import tpu_sc as plsc

N = 10000
D = 128
E = 320000

CW = 16
NC = 2
NS = 16
CHUNK = 128
G = 4
NG = 20
GE = G * CHUNK
CPT = G * NG
EPT = CHUNK * CPT
E_PAD = EPT * NC * NS
RPT = 640
NPAD = RPT * NS

NBUF = 2


def _sc_body(x, srcp, dstp, zrow, zcnt, ones_h, outa, outc,
             sidx, didx, dstc, rows, ones_v, agg_sh, cnt_sh,
             gsem, ssem, csem, isem):
    c = lax.axis_index("c")
    s = lax.axis_index("s")
    r0 = s * RPT
    pltpu.sync_copy(zrow, agg_sh.at[pl.ds(r0, RPT)])
    pltpu.sync_copy(zcnt, cnt_sh.at[pl.ds(r0, RPT)])
    pltpu.sync_copy(ones_h, ones_v)
    base = (c * NS + s) * EPT
    pltpu.sync_copy(srcp.at[pl.ds(base, GE)], sidx.at[0])
    pltpu.sync_copy(dstp.at[pl.ds(base, GE)], didx.at[0])
    plsc.subcore_barrier()
    pltpu.async_copy(x.at[sidx.at[0, pl.ds(0, CHUNK)]], rows.at[0],
                     gsem.at[0])
    pltpu.async_copy(srcp.at[pl.ds(base + GE, GE)], sidx.at[1], isem)
    pltpu.async_copy(dstp.at[pl.ds(base + GE, GE)], didx.at[1], isem)

    def _drain(b):
        pltpu.make_async_copy(rows.at[b], agg_sh.at[dstc.at[b]],
                              ssem.at[b]).wait()
        pltpu.make_async_copy(ones_v, cnt_sh.at[dstc.at[b]],
                              csem.at[b]).wait()

    def group(g, carry):
        p = lax.rem(g, 2)
        q = lax.rem(g + 1, 2)
        for j in range(G):
            i = g * G + j
            buf = lax.rem(i, NBUF)
            nxt = lax.rem(i + 1, NBUF)

            if j + 1 < G:
                @pl.when(i >= 1)
                def _():
                    _drain(nxt)

                pltpu.async_copy(
                    x.at[sidx.at[p, pl.ds((j + 1) * CHUNK, CHUNK)]],
                    rows.at[nxt], gsem.at[nxt])
            else:
                @pl.when(g + 1 < NG)
                def _():
                    _drain(nxt)
                    pltpu.make_async_copy(srcp.at[pl.ds(base, GE)],
                                          sidx.at[q], isem).wait()
                    pltpu.make_async_copy(dstp.at[pl.ds(base, GE)],
                                          didx.at[q], isem).wait()
                    pltpu.async_copy(x.at[sidx.at[q, pl.ds(0, CHUNK)]],
                                     rows.at[nxt], gsem.at[nxt])

            for k in range(CHUNK // 16):
                dstc[buf, pl.ds(k * 16, 16)] = (
                    didx[p, pl.ds(j * CHUNK + k * 16, 16)])

            pltpu.make_async_copy(
                x.at[sidx.at[p, pl.ds(j * CHUNK, CHUNK)]],
                rows.at[buf], gsem.at[buf]).wait()
            pltpu.async_copy(rows.at[buf], agg_sh.at[dstc.at[buf]],
                             ssem.at[buf], add=True)
            pltpu.async_copy(ones_v, cnt_sh.at[dstc.at[buf]],
                             csem.at[buf], add=True)

        @pl.when(g + 2 < NG)
        def _():
            off = base + (g + 2) * GE
            pltpu.async_copy(srcp.at[pl.ds(off, GE)], sidx.at[p], isem)
            pltpu.async_copy(dstp.at[pl.ds(off, GE)], didx.at[p], isem)

        return carry

    lax.fori_loop(0, NG, group, 0)
    for b in range(NBUF):
        _drain(b)
    plsc.subcore_barrier()
    pltpu.sync_copy(agg_sh.at[pl.ds(r0, RPT)],
                    outa.at[pl.ds(c * NPAD + r0, RPT)])
    pltpu.sync_copy(cnt_sh.at[pl.ds(r0, RPT)],
                    outc.at[pl.ds(c * NPAD + r0, RPT)])


_sc_call = pl.kernel(
    _sc_body,
    out_type=(
        jax.ShapeDtypeStruct((NC * NPAD, D), jnp.float32),
        jax.ShapeDtypeStruct((NC * NPAD, CW), jnp.float32),
    ),
    mesh=plsc.VectorSubcoreMesh(core_axis_name="c", subcore_axis_name="s"),
    scratch_types=[
        pltpu.VMEM((2, GE), jnp.int32),
        pltpu.VMEM((2, GE), jnp.int32),
        pltpu.VMEM((NBUF, CHUNK), jnp.int32),
        pltpu.VMEM((NBUF, CHUNK, D), jnp.float32),
        pltpu.VMEM((CHUNK, CW), jnp.float32),
        pltpu.VMEM_SHARED((NPAD, D), jnp.float32),
        pltpu.VMEM_SHARED((NPAD, CW), jnp.float32),
        pltpu.SemaphoreType.DMA((NBUF,)),
        pltpu.SemaphoreType.DMA((NBUF,)),
        pltpu.SemaphoreType.DMA((NBUF,)),
        pltpu.SemaphoreType.DMA,
    ],
    compiler_params=pltpu.CompilerParams(use_tc_tiling_on_sc=False),
)


BN = 1024
NBLK = NPAD // BN


def _tc_body(a0, a1, c0, c1, wt, bb, out_ref):
    p = a0[...] + a1[...]
    cnt = c0[:, 0:1] + c1[:, 0:1]
    num = jnp.dot(p, wt[...], preferred_element_type=jnp.float32)
    num = num + cnt * bb[...]
    out_ref[...] = jnp.maximum(num / jnp.maximum(cnt, 1.0), 0.0)


def kernel(x, edge_index, W, b):
    src = edge_index[0]
    dst = edge_index[1]
    pad = E_PAD - E
    srcp = jnp.concatenate([src, jnp.zeros((pad,), jnp.int32)])
    dstp = jnp.concatenate([dst, jnp.full((pad,), N, jnp.int32)])
    zrow = jnp.zeros((RPT, D), jnp.float32)
    zcnt = jnp.zeros((RPT, CW), jnp.float32)
    ones_h = jnp.ones((CHUNK, CW), jnp.float32)

    agg, cnt = _sc_call(x, srcp, dstp, zrow, zcnt, ones_h)

    wt = W.T
    bb = b.reshape(1, D)
    out = pl.pallas_call(
        _tc_body,
        grid=(NBLK,),
        in_specs=[
            pl.BlockSpec((BN, D), lambda i: (i, 0)),
            pl.BlockSpec((BN, D), lambda i: (i + NBLK, 0)),
            pl.BlockSpec((BN, CW), lambda i: (i, 0)),
            pl.BlockSpec((BN, CW), lambda i: (i + NBLK, 0)),
            pl.BlockSpec((D, D), lambda i: (0, 0)),
            pl.BlockSpec((1, D), lambda i: (0, 0)),
        ],
        out_specs=pl.BlockSpec((BN, D), lambda i: (i, 0)),
        out_shape=jax.ShapeDtypeStruct((N, D), jnp.float32),
    )(agg, agg, cnt, cnt, wt, bb)
    return out
